# baseline (speedup 1.0000x reference)
import functools

import jax
import jax.numpy as jnp
from jax import lax
from jax.experimental import pallas as pl
from jax.experimental.pallas import tpu as pltpu
from jax.experimental.pallas import tpu_sc as plsc

N = 10000
H = 256
G = 64
NC = 2
NS = 16
K = 64
CB = 40
NBUF = 4

ACC_ROWS = 10112


@functools.partial(jax.jit, static_argnames=("dh", "n_chunks", "feat_split"))
def _sc_segsum(ha, hb, src_r, dst_r, zeros_half, *, dh, n_chunks, feat_split):
    mesh = plsc.VectorSubcoreMesh(core_axis_name="c", subcore_axis_name="s")
    zrows = ACC_ROWS // NS

    @functools.partial(
        pl.kernel,
        out_type=jax.ShapeDtypeStruct((NC, ACC_ROWS, dh), jnp.float32),
        mesh=mesh,
        scratch_types=[
            pltpu.VMEM_SHARED((ACC_ROWS, dh), jnp.float32),
            pltpu.VMEM((CB, K), jnp.int32),
            pltpu.VMEM((CB, K), jnp.int32),
        ] + [pltpu.VMEM((K, dh), jnp.float32)] * NBUF
          + [pltpu.SemaphoreType.DMA] * (2 * NBUF),
    )
    def body(ha_hbm, hb_hbm, src_hbm, dst_hbm, z_hbm, out_hbm,
             acc, src_idx, dst_idx, *bufs_sems):
        rows = bufs_sems[:NBUF]
        sg = bufs_sems[NBUF : 2 * NBUF]
        ss = bufs_sems[2 * NBUF :]
        c = lax.axis_index("c")
        s = lax.axis_index("s")

        pltpu.sync_copy(z_hbm.at[pl.ds(s * zrows, zrows)],
                        acc.at[pl.ds(s * zrows, zrows)])
        plsc.subcore_barrier()

        def load_idx(bi):
            if feat_split:
                pltpu.sync_copy(src_hbm.at[s, pl.ds(bi * CB, CB)], src_idx)
                pltpu.sync_copy(dst_hbm.at[s, pl.ds(bi * CB, CB)], dst_idx)
            else:
                pltpu.sync_copy(src_hbm.at[c, s, pl.ds(bi * CB, CB)], src_idx)
                pltpu.sync_copy(dst_hbm.at[c, s, pl.ds(bi * CB, CB)], dst_idx)

        def run(h_hbm):
            nq = CB // 4

            def g(j, t):
                pltpu.async_copy(h_hbm.at[src_idx.at[j]], rows[t], sg[t])

            def wg(j, t):
                pltpu.make_async_copy(
                    h_hbm.at[src_idx.at[j]], rows[t], sg[t]).wait()

            def sc(j, t):
                pltpu.async_copy(rows[t], acc.at[dst_idx.at[j]], ss[t],
                                 add=True)

            def ws(t):
                pltpu.make_async_copy(
                    rows[t], acc.at[dst_idx.at[0]], ss[t]).wait()

            def blk(bi, _):
                load_idx(bi)
                g(0, 0)
                g(1, 1)

                def quad(q, _):
                    j0 = 4 * q
                    wg(j0, 0)
                    sc(j0, 0)

                    @pl.when(q > 0)
                    def _():
                        ws(2)

                    g(j0 + 2, 2)
                    wg(j0 + 1, 1)
                    sc(j0 + 1, 1)

                    @pl.when(q > 0)
                    def _():
                        ws(3)

                    g(j0 + 3, 3)
                    wg(j0 + 2, 2)
                    sc(j0 + 2, 2)

                    @pl.when(q < nq - 1)
                    def _():
                        ws(0)
                        g(j0 + 4, 0)

                    wg(j0 + 3, 3)
                    sc(j0 + 3, 3)

                    @pl.when(q < nq - 1)
                    def _():
                        ws(1)
                        g(j0 + 5, 1)

                    return 0

                lax.fori_loop(0, nq, quad, 0)
                ws(0)
                ws(1)
                ws(2)
                ws(3)
                return 0

            lax.fori_loop(0, n_chunks // CB, blk, 0)

        if feat_split:
            @pl.when(c == 0)
            def _():
                run(ha_hbm)

            @pl.when(c == 1)
            def _():
                run(hb_hbm)
        else:
            run(ha_hbm)

        plsc.subcore_barrier()
        pltpu.sync_copy(acc.at[pl.ds(s * zrows, zrows)],
                        out_hbm.at[c, pl.ds(s * zrows, zrows)])

    return body(ha, hb, src_r, dst_r, zeros_half)


def _tc_layer(ha, hb, agg, p, feat_split):
    def body(ha_ref, hb_ref, agg_ref, w1, b1, w2, b2, gamma, beta, oa, ob):
        a = agg_ref[...]
        if feat_split:
            m = jnp.concatenate([ha_ref[...] + a[0, :N],
                                 hb_ref[...] + a[1, :N]], axis=1)
        else:
            m = (jnp.concatenate([ha_ref[...], hb_ref[...]], axis=1)
                 + a[0, :N] + a[1, :N])
        z = jnp.maximum(
            jax.lax.dot_general(m, w1[...], (((1,), (0,)), ((), ())),
                                preferred_element_type=jnp.float32)
            + b1[...], 0.0)
        m2 = jax.lax.dot_general(z, w2[...], (((1,), (0,)), ((), ())),
                                 preferred_element_type=jnp.float32) + b2[...]
        mu = jnp.mean(m2, axis=0, keepdims=True)
        var = jnp.mean((m2 - mu) * (m2 - mu), axis=0, keepdims=True)
        o = jnp.maximum(
            (m2 - mu) * lax.rsqrt(var + 1e-5) * gamma[...] + beta[...], 0.0)
        oa[...] = o[:, : H // 2]
        ob[...] = o[:, H // 2 :]

    return pl.pallas_call(
        body,
        out_shape=(jax.ShapeDtypeStruct((N, H // 2), jnp.float32),
                   jax.ShapeDtypeStruct((N, H // 2), jnp.float32)),
    )(ha, hb, agg, p["W1"], p["b1"][None, :], p["W2"], p["b2"][None, :],
      p["gamma"][None, :], p["beta"][None, :])


def _tc_final_body(ha, hb, bm_ref, gf_ref, wm, wx, wg, bh1, wh2, bh2,
                   out_ref, maxacc):
    h = jnp.concatenate([ha[...], hb[...]], axis=1)
    bm = bm_ref[...]
    ids = lax.broadcasted_iota(jnp.int32, (1, G), 1)
    maskf = (bm == ids).astype(jnp.float32)
    counts = jnp.sum(maskf, axis=0, keepdims=True)
    sums = jax.lax.dot_general(maskf, h, (((0,), (0,)), ((), ())),
                               preferred_element_type=jnp.float32)
    mean_pool = sums / jnp.maximum(counts, 1.0).reshape(G, 1)

    def gbody(g, _):
        sel = jnp.where(bm == g, h, -jnp.inf)
        maxacc[pl.ds(g, 1), :] = jnp.max(sel, axis=0, keepdims=True)
        return 0

    lax.fori_loop(0, G, gbody, 0, unroll=False)
    mx = maxacc[...]
    max_pool = jnp.where(jnp.isfinite(mx), mx, 0.0)

    gf = gf_ref[...]
    hid = (jax.lax.dot_general(mean_pool, wm[...], (((1,), (0,)), ((), ())),
                               preferred_element_type=jnp.float32)
           + jax.lax.dot_general(max_pool, wx[...], (((1,), (0,)), ((), ())),
                                 preferred_element_type=jnp.float32)
           + gf[:, 0:1] * wg[0:1, :]
           + gf[:, 1:2] * wg[1:2, :]
           + gf[:, 2:3] * wg[2:3, :]
           + bh1[...])
    hid = jnp.maximum(hid, 0.0)
    out_ref[...] = jax.lax.dot_general(
        hid, wh2[...], (((1,), (0,)), ((), ())),
        preferred_element_type=jnp.float32) + bh2[...]


def _tc_final(ha, hb, batch, gf, params):
    wh1 = params["Wh1"]
    out = pl.pallas_call(
        _tc_final_body,
        out_shape=jax.ShapeDtypeStruct((G, 1), jnp.float32),
        scratch_shapes=[pltpu.VMEM((G, H), jnp.float32)],
    )(ha, hb, batch[:, None], gf, wh1[:H], wh1[H : 2 * H], wh1[2 * H :],
      params["bh1"][None, :], params["Wh2"], params["bh2"][None, :])
    return out[:, 0]


def _pad_edges(src, dst, n_workers, n_chunks):
    e = src.shape[0]
    e_pad = n_workers * n_chunks * K
    if e_pad > e:
        pad = e_pad - e
        src = jnp.concatenate([src, jnp.zeros((pad,), jnp.int32)])
        dst = jnp.concatenate([dst, jnp.full((pad,), N, jnp.int32)])
    shape = ((NC, NS, n_chunks, K) if n_workers == NC * NS
             else (NS, n_chunks, K))
    return src.reshape(shape), dst.reshape(shape)


def kernel(x, edge_index, batch, global_feats, params):
    e = edge_index.shape[1]
    src, dst = edge_index[0], edge_index[1]
    cdiv = lambda a, b: -(-a // b)
    nc_a = cdiv(cdiv(cdiv(e, NC * NS), K), CB) * CB
    src_a, dst_a = _pad_edges(src, dst, NC * NS, nc_a)
    nc_b = cdiv(cdiv(cdiv(e, NS), K), CB) * CB
    src_b, dst_b = _pad_edges(src, dst, NS, nc_b)

    dh = H // 2
    z0 = jnp.zeros((ACC_ROWS, x.shape[1]), jnp.float32)
    z1 = jnp.zeros((ACC_ROWS, dh), jnp.float32)

    layers = params["layers"]
    agg = _sc_segsum(x, x, src_a, dst_a, z0,
                     dh=x.shape[1], n_chunks=nc_a, feat_split=False)
    ha, hb = _tc_layer(x[:, : x.shape[1] // 2], x[:, x.shape[1] // 2 :],
                       agg, layers[0], feat_split=False)
    for p in layers[1:]:
        agg = _sc_segsum(ha, hb, src_b, dst_b, z1,
                         dh=dh, n_chunks=nc_b, feat_split=True)
        ha, hb = _tc_layer(ha, hb, agg, p, feat_split=True)

    return _tc_final(ha, hb, batch, global_feats, params)


# speedup vs baseline: 1.1187x; 1.1187x over previous
import functools

import jax
import jax.numpy as jnp
from jax import lax
from jax.experimental import pallas as pl
from jax.experimental.pallas import tpu as pltpu
from jax.experimental.pallas import tpu_sc as plsc

N = 10000
H = 256
G = 64
NC = 2
NS = 16
K = 128
CB = 40
NBUF = 2

ACC_ROWS = 10112


@functools.partial(jax.jit, static_argnames=("dh", "n_chunks", "feat_split"))
def _sc_segsum(ha, hb, src_r, dst_r, zeros_half, *, dh, n_chunks, feat_split):
    mesh = plsc.VectorSubcoreMesh(core_axis_name="c", subcore_axis_name="s")
    zrows = ACC_ROWS // NS

    @functools.partial(
        pl.kernel,
        out_type=jax.ShapeDtypeStruct((NC, ACC_ROWS, dh), jnp.float32),
        mesh=mesh,
        scratch_types=[
            pltpu.VMEM_SHARED((ACC_ROWS, dh), jnp.float32),
            pltpu.VMEM((CB, K), jnp.int32),
            pltpu.VMEM((CB, K), jnp.int32),
        ] + [pltpu.VMEM((K, dh), jnp.float32)] * NBUF
          + [pltpu.SemaphoreType.DMA] * (2 * NBUF),
    )
    def body(ha_hbm, hb_hbm, src_hbm, dst_hbm, z_hbm, out_hbm,
             acc, src_idx, dst_idx, *bufs_sems):
        rows = bufs_sems[:NBUF]
        sg = bufs_sems[NBUF : 2 * NBUF]
        ss = bufs_sems[2 * NBUF :]
        c = lax.axis_index("c")
        s = lax.axis_index("s")

        pltpu.sync_copy(z_hbm.at[pl.ds(s * zrows, zrows)],
                        acc.at[pl.ds(s * zrows, zrows)])
        plsc.subcore_barrier()

        def load_idx(bi):
            if feat_split:
                pltpu.sync_copy(src_hbm.at[s, pl.ds(bi * CB, CB)], src_idx)
                pltpu.sync_copy(dst_hbm.at[s, pl.ds(bi * CB, CB)], dst_idx)
            else:
                pltpu.sync_copy(src_hbm.at[c, s, pl.ds(bi * CB, CB)], src_idx)
                pltpu.sync_copy(dst_hbm.at[c, s, pl.ds(bi * CB, CB)], dst_idx)

        def run(h_hbm):
            def g(j, t):
                pltpu.async_copy(h_hbm.at[src_idx.at[j]], rows[t], sg[t])

            def wg(j, t):
                pltpu.make_async_copy(
                    h_hbm.at[src_idx.at[j]], rows[t], sg[t]).wait()

            def sc(j, t):
                pltpu.async_copy(rows[t], acc.at[dst_idx.at[j]], ss[t],
                                 add=True)

            def ws(t):
                pltpu.make_async_copy(
                    rows[t], acc.at[dst_idx.at[0]], ss[t]).wait()

            def blk(bi, _):
                load_idx(bi)
                g(0, 0)

                def pair(jj, _):
                    j0 = 2 * jj
                    j1 = j0 + 1

                    @pl.when(jj > 0)
                    def _():
                        ws(1)

                    g(j1, 1)
                    wg(j0, 0)
                    sc(j0, 0)

                    @pl.when(j1 + 1 < CB)
                    def _():
                        ws(0)
                        g(j1 + 1, 0)

                    wg(j1, 1)
                    sc(j1, 1)
                    return 0

                lax.fori_loop(0, CB // 2, pair, 0)
                ws(0)
                ws(1)
                return 0

            lax.fori_loop(0, n_chunks // CB, blk, 0)

        if feat_split:
            @pl.when(c == 0)
            def _():
                run(ha_hbm)

            @pl.when(c == 1)
            def _():
                run(hb_hbm)
        else:
            run(ha_hbm)

        plsc.subcore_barrier()
        pltpu.sync_copy(acc.at[pl.ds(s * zrows, zrows)],
                        out_hbm.at[c, pl.ds(s * zrows, zrows)])

    return body(ha, hb, src_r, dst_r, zeros_half)


def _tc_layer(ha, hb, agg, p, feat_split):
    def body(ha_ref, hb_ref, agg_ref, w1, b1, w2, b2, gamma, beta, oa, ob):
        a = agg_ref[...]
        if feat_split:
            m = jnp.concatenate([ha_ref[...] + a[0, :N],
                                 hb_ref[...] + a[1, :N]], axis=1)
        else:
            m = (jnp.concatenate([ha_ref[...], hb_ref[...]], axis=1)
                 + a[0, :N] + a[1, :N])
        z = jnp.maximum(
            jax.lax.dot_general(m, w1[...], (((1,), (0,)), ((), ())),
                                preferred_element_type=jnp.float32)
            + b1[...], 0.0)
        m2 = jax.lax.dot_general(z, w2[...], (((1,), (0,)), ((), ())),
                                 preferred_element_type=jnp.float32) + b2[...]
        mu = jnp.mean(m2, axis=0, keepdims=True)
        var = jnp.mean((m2 - mu) * (m2 - mu), axis=0, keepdims=True)
        o = jnp.maximum(
            (m2 - mu) * lax.rsqrt(var + 1e-5) * gamma[...] + beta[...], 0.0)
        oa[...] = o[:, : H // 2]
        ob[...] = o[:, H // 2 :]

    return pl.pallas_call(
        body,
        out_shape=(jax.ShapeDtypeStruct((N, H // 2), jnp.float32),
                   jax.ShapeDtypeStruct((N, H // 2), jnp.float32)),
    )(ha, hb, agg, p["W1"], p["b1"][None, :], p["W2"], p["b2"][None, :],
      p["gamma"][None, :], p["beta"][None, :])


def _tc_final_body(ha, hb, bm_ref, gf_ref, wm, wx, wg, bh1, wh2, bh2,
                   out_ref, maxacc):
    h = jnp.concatenate([ha[...], hb[...]], axis=1)
    bm = bm_ref[...]
    ids = lax.broadcasted_iota(jnp.int32, (1, G), 1)
    maskf = (bm == ids).astype(jnp.float32)
    counts = jnp.sum(maskf, axis=0, keepdims=True)
    sums = jax.lax.dot_general(maskf, h, (((0,), (0,)), ((), ())),
                               preferred_element_type=jnp.float32)
    mean_pool = sums / jnp.maximum(counts, 1.0).reshape(G, 1)

    def gbody(g, _):
        sel = jnp.where(bm == g, h, -jnp.inf)
        maxacc[pl.ds(g, 1), :] = jnp.max(sel, axis=0, keepdims=True)
        return 0

    lax.fori_loop(0, G, gbody, 0, unroll=False)
    mx = maxacc[...]
    max_pool = jnp.where(jnp.isfinite(mx), mx, 0.0)

    gf = gf_ref[...]
    hid = (jax.lax.dot_general(mean_pool, wm[...], (((1,), (0,)), ((), ())),
                               preferred_element_type=jnp.float32)
           + jax.lax.dot_general(max_pool, wx[...], (((1,), (0,)), ((), ())),
                                 preferred_element_type=jnp.float32)
           + gf[:, 0:1] * wg[0:1, :]
           + gf[:, 1:2] * wg[1:2, :]
           + gf[:, 2:3] * wg[2:3, :]
           + bh1[...])
    hid = jnp.maximum(hid, 0.0)
    out_ref[...] = jax.lax.dot_general(
        hid, wh2[...], (((1,), (0,)), ((), ())),
        preferred_element_type=jnp.float32) + bh2[...]


def _tc_final(ha, hb, batch, gf, params):
    wh1 = params["Wh1"]
    out = pl.pallas_call(
        _tc_final_body,
        out_shape=jax.ShapeDtypeStruct((G, 1), jnp.float32),
        scratch_shapes=[pltpu.VMEM((G, H), jnp.float32)],
    )(ha, hb, batch[:, None], gf, wh1[:H], wh1[H : 2 * H], wh1[2 * H :],
      params["bh1"][None, :], params["Wh2"], params["bh2"][None, :])
    return out[:, 0]


def _pad_edges(src, dst, n_workers, n_chunks):
    e = src.shape[0]
    e_pad = n_workers * n_chunks * K
    if e_pad > e:
        pad = e_pad - e
        src = jnp.concatenate([src, jnp.zeros((pad,), jnp.int32)])
        dst = jnp.concatenate([dst, jnp.full((pad,), N, jnp.int32)])
    shape = ((NC, NS, n_chunks, K) if n_workers == NC * NS
             else (NS, n_chunks, K))
    return src.reshape(shape), dst.reshape(shape)


def kernel(x, edge_index, batch, global_feats, params):
    e = edge_index.shape[1]
    src, dst = edge_index[0], edge_index[1]
    cdiv = lambda a, b: -(-a // b)
    nc_a = cdiv(cdiv(cdiv(e, NC * NS), K), CB) * CB
    src_a, dst_a = _pad_edges(src, dst, NC * NS, nc_a)
    nc_b = cdiv(cdiv(cdiv(e, NS), K), CB) * CB
    src_b, dst_b = _pad_edges(src, dst, NS, nc_b)

    dh = H // 2
    z0 = jnp.zeros((ACC_ROWS, x.shape[1]), jnp.float32)
    z1 = jnp.zeros((ACC_ROWS, dh), jnp.float32)

    layers = params["layers"]
    agg = _sc_segsum(x, x, src_a, dst_a, z0,
                     dh=x.shape[1], n_chunks=nc_a, feat_split=False)
    ha, hb = _tc_layer(x[:, : x.shape[1] // 2], x[:, x.shape[1] // 2 :],
                       agg, layers[0], feat_split=False)
    for p in layers[1:]:
        agg = _sc_segsum(ha, hb, src_b, dst_b, z1,
                         dh=dh, n_chunks=nc_b, feat_split=True)
        ha, hb = _tc_layer(ha, hb, agg, p, feat_split=True)

    return _tc_final(ha, hb, batch, global_feats, params)
